# baseline (speedup 1.0000x reference)
import functools

import jax
import jax.numpy as jnp
from jax import lax
from jax.experimental import pallas as pl
from jax.experimental.pallas import tpu as pltpu
from jax.experimental.pallas import tpu_sc as plsc

_B = 4096
_TOTAL_BITS = 1024
_NUM_NEURONS = 1024
_N_BITS = 14
_MEM_SIZE = 1 << _N_BITS

_BM = 512
_NW = 32
_FLAT = _B * _NUM_NEURONS
_PER_W = _FLAT // _NW
_CH = 16384
_NCH = _PER_W // _CH


def _addr_body(bits_ref, conn_ref, out_ref, whi_ref, wlo_ref, nb_ref):
    @pl.when(pl.program_id(0) == 0)
    def _build_w():
        t = lax.broadcasted_iota(jnp.int32, (_NUM_NEURONS, _TOTAL_BITS), 1)
        hi = jnp.zeros((_NUM_NEURONS, _TOTAL_BITS), jnp.float32)
        lo = jnp.zeros((_NUM_NEURONS, _TOTAL_BITS), jnp.float32)
        for k in range(7):
            c = conn_ref[:, k : k + 1]
            hi = hi + jnp.where(t == c, jnp.float32(1024 << (6 - k)), 0.0)
        for k in range(7, _N_BITS):
            c = conn_ref[:, k : k + 1]
            lo = lo + jnp.where(t == c, jnp.float32(1 << (_N_BITS - 1 - k)), 0.0)
        whi_ref[:] = hi.astype(jnp.bfloat16)
        wlo_ref[:] = lo.astype(jnp.bfloat16)
        nn = lax.broadcasted_iota(jnp.int32, (_NUM_NEURONS, 128), 0)
        nb_ref[:] = (nn >> 3) * (_MEM_SIZE * 8) + (nn & 7) * 128

    bits = (bits_ref[:] != 0).astype(jnp.bfloat16)
    bt = bits.T
    hi = jnp.dot(whi_ref[:], bt, preferred_element_type=jnp.float32)
    lo = jnp.dot(wlo_ref[:], bt, preferred_element_type=jnp.float32)
    out_ref[:] = (hi + lo).astype(jnp.int32) + nb_ref[:, 0:1]


def _addresses_t(input_bits, conn_p):
    return pl.pallas_call(
        _addr_body,
        grid=(_B // _BM,),
        in_specs=[
            pl.BlockSpec((_BM, _TOTAL_BITS), lambda i: (i, 0)),
            pl.BlockSpec((_NUM_NEURONS, 16), lambda i: (0, 0)),
        ],
        out_specs=pl.BlockSpec((_NUM_NEURONS, _BM), lambda i: (0, i)),
        out_shape=jax.ShapeDtypeStruct((_NUM_NEURONS, _B), jnp.int32),
        scratch_shapes=[
            pltpu.VMEM((_NUM_NEURONS, _TOTAL_BITS), jnp.bfloat16),
            pltpu.VMEM((_NUM_NEURONS, _TOTAL_BITS), jnp.bfloat16),
            pltpu.VMEM((_NUM_NEURONS, 128), jnp.int32),
        ],
    )(input_bits, conn_p)


def _tr_body(in_ref, out_ref):
    out_ref[:] = in_ref[:].T


def _transpose_nb_to_bn(out_t):
    return pl.pallas_call(
        _tr_body,
        grid=(_B // _BM,),
        in_specs=[pl.BlockSpec((_NUM_NEURONS, _BM), lambda i: (0, i))],
        out_specs=pl.BlockSpec((_BM, _NUM_NEURONS), lambda i: (i, 0)),
        out_shape=jax.ShapeDtypeStruct((_B, _NUM_NEURONS), jnp.float32),
    )(out_t)


@functools.lru_cache(maxsize=1)
def _make_gather():
    mesh = plsc.VectorSubcoreMesh(core_axis_name="c", subcore_axis_name="s")

    @functools.partial(
        pl.kernel,
        mesh=mesh,
        out_type=jax.ShapeDtypeStruct((_FLAT,), jnp.float32),
        scratch_types=[
            pltpu.VMEM((_CH,), jnp.int32),
            pltpu.VMEM((_CH,), jnp.int32),
            pltpu.VMEM((_CH,), jnp.float32),
            pltpu.VMEM((_CH,), jnp.float32),
            pltpu.SemaphoreType.DMA,
            pltpu.SemaphoreType.DMA,
            pltpu.SemaphoreType.DMA,
        ],
    )
    def gather_k(mem_hbm, idx_hbm, out_hbm, idx_v0, idx_v1, val_v0, val_v1,
                 gsem0, gsem1, wsem):
        wid = lax.axis_index("s") * 2 + lax.axis_index("c")
        base = wid * _PER_W
        idx_bufs = (idx_v0, idx_v1)
        val_bufs = (val_v0, val_v1)
        half = _CH // 2

        def start_gather(buf):
            g0 = pltpu.async_copy(
                mem_hbm.at[idx_bufs[buf].at[pl.ds(0, half)]],
                val_bufs[buf].at[pl.ds(0, half)], gsem0,
            )
            g1 = pltpu.async_copy(
                mem_hbm.at[idx_bufs[buf].at[pl.ds(half, half)]],
                val_bufs[buf].at[pl.ds(half, half)], gsem1,
            )
            return g0, g1

        pltpu.sync_copy(idx_hbm.at[pl.ds(base, _CH)], idx_v0)
        gathers = [None, None]
        wbs = [None, None]
        gathers[0] = start_gather(0)
        for c in range(_NCH):
            cur = c % 2
            nxt = (c + 1) % 2
            if c + 1 < _NCH:
                pltpu.sync_copy(
                    idx_hbm.at[pl.ds(base + (c + 1) * _CH, _CH)], idx_bufs[nxt]
                )
            gathers[cur][0].wait()
            gathers[cur][1].wait()
            wbs[cur] = pltpu.async_copy(
                val_bufs[cur], out_hbm.at[pl.ds(base + c * _CH, _CH)], wsem
            )
            if c + 1 < _NCH:
                if wbs[nxt] is not None:
                    wbs[nxt].wait()
                gathers[nxt] = start_gather(nxt)
        wbs[(_NCH - 1) % 2].wait()

    return gather_k


def kernel(input_bits, connections, memory):
    conn_p = jnp.full((_NUM_NEURONS, 16), _TOTAL_BITS, jnp.int32)
    conn_p = conn_p.at[:, :_N_BITS].set(connections)
    idx_t = _addresses_t(input_bits, conn_p)
    mem_lin = (
        memory.reshape(_NUM_NEURONS // 8, 8, _MEM_SIZE // 128, 128)
        .transpose(0, 2, 1, 3)
        .reshape(_NUM_NEURONS * _MEM_SIZE)
    )
    idx_flat = (
        idx_t.reshape(_NUM_NEURONS // 8, 8, _B // 128, 128)
        .transpose(0, 2, 1, 3)
        .reshape(_FLAT)
    )
    out_flat = _make_gather()(mem_lin, idx_flat)
    out_t = (
        out_flat.reshape(_NUM_NEURONS // 8, _B // 128, 8, 128)
        .transpose(0, 2, 1, 3)
        .reshape(_NUM_NEURONS, _B)
    )
    return _transpose_nb_to_bn(out_t)


# speedup vs baseline: 1.0630x; 1.0630x over previous
import functools

import jax
import jax.numpy as jnp
from jax import lax
from jax.experimental import pallas as pl
from jax.experimental.pallas import tpu as pltpu
from jax.experimental.pallas import tpu_sc as plsc

_B = 4096
_TOTAL_BITS = 1024
_NUM_NEURONS = 1024
_N_BITS = 14
_MEM_SIZE = 1 << _N_BITS

_BM = 1024
_NW = 32
_FLAT = _B * _NUM_NEURONS
_PER_W = _FLAT // _NW
_CH = 16384
_NCH = _PER_W // _CH


def _addr_body(bits_ref, connt_ref, out_ref, whi_ref, wlo_ref, nb_ref):
    @pl.when(pl.program_id(0) == 0)
    def _build_w():
        t = lax.broadcasted_iota(jnp.int32, (_TOTAL_BITS, _NUM_NEURONS), 0)
        hi = jnp.zeros((_TOTAL_BITS, _NUM_NEURONS), jnp.float32)
        lo = jnp.zeros((_TOTAL_BITS, _NUM_NEURONS), jnp.float32)
        for k in range(7):
            c = connt_ref[k : k + 1, :]
            hi = hi + jnp.where(t == c, jnp.float32(1024 << (6 - k)), 0.0)
        for k in range(7, _N_BITS):
            c = connt_ref[k : k + 1, :]
            lo = lo + jnp.where(t == c, jnp.float32(1 << (_N_BITS - 1 - k)), 0.0)
        whi_ref[:] = hi.astype(jnp.bfloat16)
        wlo_ref[:] = lo.astype(jnp.bfloat16)
        nn = lax.broadcasted_iota(jnp.int32, (8, _NUM_NEURONS), 1)
        nb_ref[:] = (nn >> 3) * (_MEM_SIZE * 8) + (nn & 7) * 128

    bits = (bits_ref[:] != 0).astype(jnp.bfloat16)
    hi = jnp.dot(bits, whi_ref[:], preferred_element_type=jnp.float32)
    lo = jnp.dot(bits, wlo_ref[:], preferred_element_type=jnp.float32)
    phys = (hi + lo).astype(jnp.int32) + nb_ref[0:1, :]
    out_ref[:] = (
        phys.reshape(_BM // 8, 8, _NUM_NEURONS // 128, 128)
        .transpose(0, 2, 1, 3)
        .reshape(_BM * _NUM_NEURONS)
    )


def _addresses(input_bits, connt):
    return pl.pallas_call(
        _addr_body,
        grid=(_B // _BM,),
        in_specs=[
            pl.BlockSpec((_BM, _TOTAL_BITS), lambda i: (i, 0)),
            pl.BlockSpec((16, _NUM_NEURONS), lambda i: (0, 0)),
        ],
        out_specs=pl.BlockSpec((_BM * _NUM_NEURONS,), lambda i: (i,)),
        out_shape=jax.ShapeDtypeStruct((_FLAT,), jnp.int32),
        scratch_shapes=[
            pltpu.VMEM((_TOTAL_BITS, _NUM_NEURONS), jnp.bfloat16),
            pltpu.VMEM((_TOTAL_BITS, _NUM_NEURONS), jnp.bfloat16),
            pltpu.VMEM((8, _NUM_NEURONS), jnp.int32),
        ],
    )(input_bits, connt)


@functools.lru_cache(maxsize=1)
def _make_gather():
    mesh = plsc.VectorSubcoreMesh(core_axis_name="c", subcore_axis_name="s")

    @functools.partial(
        pl.kernel,
        mesh=mesh,
        out_type=jax.ShapeDtypeStruct((_FLAT,), jnp.float32),
        scratch_types=[
            pltpu.VMEM((_CH,), jnp.int32),
            pltpu.VMEM((_CH,), jnp.int32),
            pltpu.VMEM((_CH,), jnp.float32),
            pltpu.VMEM((_CH,), jnp.float32),
            pltpu.SemaphoreType.DMA,
            pltpu.SemaphoreType.DMA,
            pltpu.SemaphoreType.DMA,
        ],
    )
    def gather_k(mem_hbm, idx_hbm, out_hbm, idx_v0, idx_v1, val_v0, val_v1,
                 gsem0, gsem1, wsem):
        wid = lax.axis_index("s") * 2 + lax.axis_index("c")
        base = wid * _PER_W
        idx_bufs = (idx_v0, idx_v1)
        val_bufs = (val_v0, val_v1)
        half = _CH // 2

        def start_gather(buf):
            g0 = pltpu.async_copy(
                mem_hbm.at[idx_bufs[buf].at[pl.ds(0, half)]],
                val_bufs[buf].at[pl.ds(0, half)], gsem0,
            )
            g1 = pltpu.async_copy(
                mem_hbm.at[idx_bufs[buf].at[pl.ds(half, half)]],
                val_bufs[buf].at[pl.ds(half, half)], gsem1,
            )
            return g0, g1

        pltpu.sync_copy(idx_hbm.at[pl.ds(base, _CH)], idx_v0)
        gathers = [None, None]
        wbs = [None, None]
        gathers[0] = start_gather(0)
        for c in range(_NCH):
            cur = c % 2
            nxt = (c + 1) % 2
            if c + 1 < _NCH:
                pltpu.sync_copy(
                    idx_hbm.at[pl.ds(base + (c + 1) * _CH, _CH)], idx_bufs[nxt]
                )
            gathers[cur][0].wait()
            gathers[cur][1].wait()
            wbs[cur] = pltpu.async_copy(
                val_bufs[cur], out_hbm.at[pl.ds(base + c * _CH, _CH)], wsem
            )
            if c + 1 < _NCH:
                if wbs[nxt] is not None:
                    wbs[nxt].wait()
                gathers[nxt] = start_gather(nxt)
        wbs[(_NCH - 1) % 2].wait()

    return gather_k


def kernel(input_bits, connections, memory):
    connt = jnp.full((16, _NUM_NEURONS), _TOTAL_BITS, jnp.int32)
    connt = connt.at[:_N_BITS].set(connections.T)
    idx = _addresses(input_bits, connt)
    mem_lin = (
        memory.reshape(_NUM_NEURONS // 8, 8, _MEM_SIZE // 128, 128)
        .transpose(0, 2, 1, 3)
        .reshape(_NUM_NEURONS * _MEM_SIZE)
    )
    out_flat = _make_gather()(mem_lin, idx)
    return (
        out_flat.reshape(_B // 8, 8, _NUM_NEURONS // 128, 128)
        .transpose(0, 2, 1, 3)
        .reshape(_B, _NUM_NEURONS)
    )


# speedup vs baseline: 1.0678x; 1.0045x over previous
import functools

import jax
import jax.numpy as jnp
from jax import lax
from jax.experimental import pallas as pl
from jax.experimental.pallas import tpu as pltpu
from jax.experimental.pallas import tpu_sc as plsc

_B = 4096
_TOTAL_BITS = 1024
_NUM_NEURONS = 1024
_N_BITS = 14
_MEM_SIZE = 1 << _N_BITS

_BM = 512
_NW = 32
_FLAT = _B * _NUM_NEURONS
_PER_W = _FLAT // _NW
_CH = 16384
_NCH = _PER_W // _CH


def _addr_body(bits_ref, connt_ref, out_ref, whi_ref, wlo_ref, nb_ref):
    @pl.when(pl.program_id(0) == 0)
    def _build_w():
        t = lax.broadcasted_iota(jnp.int32, (_TOTAL_BITS, _NUM_NEURONS), 0)
        hi = jnp.zeros((_TOTAL_BITS, _NUM_NEURONS), jnp.float32)
        lo = jnp.zeros((_TOTAL_BITS, _NUM_NEURONS), jnp.float32)
        for k in range(7):
            c = connt_ref[k : k + 1, :]
            hi = hi + jnp.where(t == c, jnp.float32(1024 << (6 - k)), 0.0)
        for k in range(7, _N_BITS):
            c = connt_ref[k : k + 1, :]
            lo = lo + jnp.where(t == c, jnp.float32(1 << (_N_BITS - 1 - k)), 0.0)
        whi_ref[:] = hi.astype(jnp.bfloat16)
        wlo_ref[:] = lo.astype(jnp.bfloat16)
        nn = lax.broadcasted_iota(jnp.int32, (8, _NUM_NEURONS), 1)
        nb_ref[:] = (nn >> 3) * (_MEM_SIZE * 8) + (nn & 7) * 128

    bits = (bits_ref[:] != 0).astype(jnp.bfloat16)
    hi = jnp.dot(bits, whi_ref[:], preferred_element_type=jnp.float32)
    lo = jnp.dot(bits, wlo_ref[:], preferred_element_type=jnp.float32)
    phys = (hi + lo).astype(jnp.int32) + nb_ref[0:1, :]
    out_ref[:] = (
        phys.reshape(_BM // 8, 8, _NUM_NEURONS // 128, 128)
        .transpose(0, 2, 1, 3)
        .reshape(_BM * _NUM_NEURONS)
    )


def _addresses(input_bits, connt):
    return pl.pallas_call(
        _addr_body,
        grid=(_B // _BM,),
        in_specs=[
            pl.BlockSpec((_BM, _TOTAL_BITS), lambda i: (i, 0)),
            pl.BlockSpec((16, _NUM_NEURONS), lambda i: (0, 0)),
        ],
        out_specs=pl.BlockSpec((_BM * _NUM_NEURONS,), lambda i: (i,)),
        out_shape=jax.ShapeDtypeStruct((_FLAT,), jnp.int32),
        scratch_shapes=[
            pltpu.VMEM((_TOTAL_BITS, _NUM_NEURONS), jnp.bfloat16),
            pltpu.VMEM((_TOTAL_BITS, _NUM_NEURONS), jnp.bfloat16),
            pltpu.VMEM((8, _NUM_NEURONS), jnp.int32),
        ],
    )(input_bits, connt)


@functools.lru_cache(maxsize=1)
def _make_gather():
    mesh = plsc.VectorSubcoreMesh(core_axis_name="c", subcore_axis_name="s")

    @functools.partial(
        pl.kernel,
        mesh=mesh,
        out_type=jax.ShapeDtypeStruct((_FLAT,), jnp.float32),
        scratch_types=[
            pltpu.VMEM((_CH,), jnp.int32),
            pltpu.VMEM((_CH,), jnp.int32),
            pltpu.VMEM((_CH,), jnp.float32),
            pltpu.VMEM((_CH,), jnp.float32),
            pltpu.SemaphoreType.DMA,
            pltpu.SemaphoreType.DMA,
            pltpu.SemaphoreType.DMA,
        ],
    )
    def gather_k(mem_hbm, idx_hbm, out_hbm, idx_v0, idx_v1, val_v0, val_v1,
                 gsem0, gsem1, wsem):
        wid = lax.axis_index("s") * 2 + lax.axis_index("c")
        base = wid * _PER_W
        idx_bufs = (idx_v0, idx_v1)
        val_bufs = (val_v0, val_v1)
        half = _CH // 2

        def start_gather(buf):
            g0 = pltpu.async_copy(
                mem_hbm.at[idx_bufs[buf].at[pl.ds(0, half)]],
                val_bufs[buf].at[pl.ds(0, half)], gsem0,
            )
            g1 = pltpu.async_copy(
                mem_hbm.at[idx_bufs[buf].at[pl.ds(half, half)]],
                val_bufs[buf].at[pl.ds(half, half)], gsem1,
            )
            return g0, g1

        pltpu.sync_copy(idx_hbm.at[pl.ds(base, _CH)], idx_v0)
        gathers = [None, None]
        wbs = [None, None]
        gathers[0] = start_gather(0)
        for c in range(_NCH):
            cur = c % 2
            nxt = (c + 1) % 2
            if c + 1 < _NCH:
                pltpu.sync_copy(
                    idx_hbm.at[pl.ds(base + (c + 1) * _CH, _CH)], idx_bufs[nxt]
                )
            gathers[cur][0].wait()
            gathers[cur][1].wait()
            wbs[cur] = pltpu.async_copy(
                val_bufs[cur], out_hbm.at[pl.ds(base + c * _CH, _CH)], wsem
            )
            if c + 1 < _NCH:
                if wbs[nxt] is not None:
                    wbs[nxt].wait()
                gathers[nxt] = start_gather(nxt)
        wbs[(_NCH - 1) % 2].wait()

    return gather_k


def kernel(input_bits, connections, memory):
    connt = jnp.full((16, _NUM_NEURONS), _TOTAL_BITS, jnp.int32)
    connt = connt.at[:_N_BITS].set(connections.T)
    idx = _addresses(input_bits, connt)
    mem_lin = (
        memory.reshape(_NUM_NEURONS // 8, 8, _MEM_SIZE // 128, 128)
        .transpose(0, 2, 1, 3)
        .reshape(_NUM_NEURONS * _MEM_SIZE)
    )
    out_flat = _make_gather()(mem_lin, idx)
    return (
        out_flat.reshape(_B // 8, 8, _NUM_NEURONS // 128, 128)
        .transpose(0, 2, 1, 3)
        .reshape(_B, _NUM_NEURONS)
    )


# speedup vs baseline: 1.4715x; 1.3781x over previous
import functools

import jax
import jax.numpy as jnp
from jax import lax
from jax.experimental import pallas as pl
from jax.experimental.pallas import tpu as pltpu
from jax.experimental.pallas import tpu_sc as plsc

_B = 4096
_TOTAL_BITS = 1024
_NUM_NEURONS = 1024
_N_BITS = 14
_MEM_SIZE = 1 << _N_BITS

_BM = 512
_FLAT = _B * _NUM_NEURONS
_SR = 8 * _MEM_SIZE
_SLOTS_SR = 8 * _B
_CH = 16384
_NPASS = 8
_GRP = 8
_SHW = _GRP * _SR


def _addr_body(bits_ref, conn_ref, out_ref, whi_ref, wlo_ref, nb_ref):
    @pl.when(pl.program_id(0) == 0)
    def _build_w():
        t = lax.broadcasted_iota(jnp.int32, (_NUM_NEURONS, _TOTAL_BITS), 1)
        hi = jnp.zeros((_NUM_NEURONS, _TOTAL_BITS), jnp.float32)
        lo = jnp.zeros((_NUM_NEURONS, _TOTAL_BITS), jnp.float32)
        for k in range(7):
            c = conn_ref[:, k : k + 1]
            hi = hi + jnp.where(t == c, jnp.float32(1024 << (6 - k)), 0.0)
        for k in range(7, _N_BITS):
            c = conn_ref[:, k : k + 1]
            lo = lo + jnp.where(t == c, jnp.float32(1 << (_N_BITS - 1 - k)), 0.0)
        whi_ref[:] = hi.astype(jnp.bfloat16)
        wlo_ref[:] = lo.astype(jnp.bfloat16)
        nn = lax.broadcasted_iota(jnp.int32, (_NUM_NEURONS, 128), 0)
        nb_ref[:] = ((nn >> 3) & 7) * _SR + (nn & 7) * 128

    bits = (bits_ref[:] != 0).astype(jnp.bfloat16)
    bt = bits.T
    hi = jnp.dot(whi_ref[:], bt, preferred_element_type=jnp.float32)
    lo = jnp.dot(wlo_ref[:], bt, preferred_element_type=jnp.float32)
    out_ref[:] = (hi + lo).astype(jnp.int32) + nb_ref[:, 0:1]


def _addresses_t(input_bits, conn_p):
    return pl.pallas_call(
        _addr_body,
        grid=(_B // _BM,),
        in_specs=[
            pl.BlockSpec((_BM, _TOTAL_BITS), lambda i: (i, 0)),
            pl.BlockSpec((_NUM_NEURONS, 16), lambda i: (0, 0)),
        ],
        out_specs=pl.BlockSpec((_NUM_NEURONS, _BM), lambda i: (0, i)),
        out_shape=jax.ShapeDtypeStruct((_NUM_NEURONS, _B), jnp.int32),
        scratch_shapes=[
            pltpu.VMEM((_NUM_NEURONS, _TOTAL_BITS), jnp.bfloat16),
            pltpu.VMEM((_NUM_NEURONS, _TOTAL_BITS), jnp.bfloat16),
            pltpu.VMEM((_NUM_NEURONS, 128), jnp.int32),
        ],
    )(input_bits, conn_p)


def _tr_body(in_ref, out_ref):
    out_ref[:] = in_ref[:].T


def _transpose_nb_to_bn(out_t):
    return pl.pallas_call(
        _tr_body,
        grid=(_B // _BM,),
        in_specs=[pl.BlockSpec((_NUM_NEURONS, _BM), lambda i: (0, i))],
        out_specs=pl.BlockSpec((_BM, _NUM_NEURONS), lambda i: (i, 0)),
        out_shape=jax.ShapeDtypeStruct((_B, _NUM_NEURONS), jnp.float32),
    )(out_t)


@functools.lru_cache(maxsize=1)
def _make_gather():
    mesh = plsc.VectorSubcoreMesh(core_axis_name="c", subcore_axis_name="s")

    @functools.partial(
        pl.kernel,
        mesh=mesh,
        out_type=jax.ShapeDtypeStruct((_FLAT,), jnp.float32),
        scratch_types=[
            pltpu.VMEM_SHARED((_SHW,), jnp.float32),
            pltpu.VMEM((_CH,), jnp.int32),
            pltpu.VMEM((_CH,), jnp.int32),
            pltpu.VMEM((_CH,), jnp.float32),
            pltpu.VMEM((_CH,), jnp.float32),
            pltpu.SemaphoreType.DMA,
            pltpu.SemaphoreType.DMA,
            pltpu.SemaphoreType.DMA,
        ],
    )
    def gather_k(mem_hbm, idx_hbm, out_hbm, shared, idx_v0, idx_v1,
                 val_v0, val_v1, gsem0, gsem1, wsem):
        s = lax.axis_index("s")
        c = lax.axis_index("c")
        core_slot_base = c * (64 * _SLOTS_SR)
        core_mem_base = c * (64 * _SR)
        idx_bufs = (idx_v0, idx_v1)
        val_bufs = (val_v0, val_v1)
        half = _CH // 2
        wbs = [None, None]

        def chunk_off(k):
            return core_slot_base + k * (_GRP * _SLOTS_SR) + s * _CH

        pltpu.sync_copy(idx_hbm.at[pl.ds(chunk_off(0), _CH)], idx_v0)

        for k in range(_NPASS):
            cur = k % 2
            nxt = (k + 1) % 2
            pltpu.sync_copy(
                mem_hbm.at[
                    pl.ds(core_mem_base + k * _SHW + s * (_SHW // 16),
                          _SHW // 16)
                ],
                shared.at[pl.ds(s * (_SHW // 16), _SHW // 16)],
            )
            plsc.subcore_barrier()
            g0 = pltpu.async_copy(
                shared.at[idx_bufs[cur].at[pl.ds(0, half)]],
                val_bufs[cur].at[pl.ds(0, half)], gsem0,
            )
            g1 = pltpu.async_copy(
                shared.at[idx_bufs[cur].at[pl.ds(half, half)]],
                val_bufs[cur].at[pl.ds(half, half)], gsem1,
            )
            if k + 1 < _NPASS:
                pltpu.sync_copy(
                    idx_hbm.at[pl.ds(chunk_off(k + 1), _CH)], idx_bufs[nxt]
                )
                if wbs[nxt] is not None:
                    wbs[nxt].wait()
            g0.wait()
            g1.wait()
            wbs[cur] = pltpu.async_copy(
                val_bufs[cur], out_hbm.at[pl.ds(chunk_off(k), _CH)], wsem
            )
            plsc.subcore_barrier()
        wbs[0].wait()
        wbs[1].wait()

    return gather_k


def kernel(input_bits, connections, memory):
    conn_p = jnp.full((_NUM_NEURONS, 16), _TOTAL_BITS, jnp.int32)
    conn_p = conn_p.at[:, :_N_BITS].set(connections)
    idx_t = _addresses_t(input_bits, conn_p)
    mem_lin = (
        memory.reshape(_NUM_NEURONS // 8, 8, _MEM_SIZE // 128, 128)
        .transpose(0, 2, 1, 3)
        .reshape(_NUM_NEURONS * _MEM_SIZE)
    )
    idx_flat = (
        idx_t.reshape(_NUM_NEURONS // 8, 8, _B // 128, 128)
        .transpose(0, 2, 1, 3)
        .reshape(_FLAT)
    )
    out_flat = _make_gather()(mem_lin, idx_flat)
    out_t = (
        out_flat.reshape(_NUM_NEURONS // 8, _B // 128, 8, 128)
        .transpose(0, 2, 1, 3)
        .reshape(_NUM_NEURONS, _B)
    )
    return _transpose_nb_to_bn(out_t)


# speedup vs baseline: 1.9009x; 1.2918x over previous
import functools

import jax
import jax.numpy as jnp
from jax import lax
from jax.experimental import pallas as pl
from jax.experimental.pallas import tpu as pltpu
from jax.experimental.pallas import tpu_sc as plsc

_B = 4096
_TOTAL_BITS = 1024
_NUM_NEURONS = 1024
_N_BITS = 14
_MEM_SIZE = 1 << _N_BITS

_BM = 512
_FLAT = _B * _NUM_NEURONS
_SR = 8 * _MEM_SIZE
_SLOTS_SR = 8 * _B
_NPASS = 16
_GRP = 4
_SHW = _GRP * _SR
_CH = _GRP * _SLOTS_SR // 16


def _addr_body(bits_ref, conn_ref, out_ref, whi_ref, wlo_ref, nb_ref):
    @pl.when(pl.program_id(0) == 0)
    def _build_w():
        t = lax.broadcasted_iota(jnp.int32, (_NUM_NEURONS, _TOTAL_BITS), 1)
        hi = jnp.zeros((_NUM_NEURONS, _TOTAL_BITS), jnp.float32)
        lo = jnp.zeros((_NUM_NEURONS, _TOTAL_BITS), jnp.float32)
        for k in range(7):
            c = conn_ref[:, k : k + 1]
            hi = hi + jnp.where(t == c, jnp.float32(1024 << (6 - k)), 0.0)
        for k in range(7, _N_BITS):
            c = conn_ref[:, k : k + 1]
            lo = lo + jnp.where(t == c, jnp.float32(1 << (_N_BITS - 1 - k)), 0.0)
        whi_ref[:] = hi.astype(jnp.bfloat16)
        wlo_ref[:] = lo.astype(jnp.bfloat16)
        nn = lax.broadcasted_iota(jnp.int32, (_NUM_NEURONS, 128), 0)
        nb_ref[:] = ((nn >> 3) & (_GRP - 1)) * _SR + (nn & 7) * 128

    bits = (bits_ref[:] != 0).astype(jnp.bfloat16)
    bt = bits.T
    hi = jnp.dot(whi_ref[:], bt, preferred_element_type=jnp.float32)
    lo = jnp.dot(wlo_ref[:], bt, preferred_element_type=jnp.float32)
    out_ref[:] = (hi + lo).astype(jnp.int32) + nb_ref[:, 0:1]


def _addresses_t(input_bits, conn_p):
    return pl.pallas_call(
        _addr_body,
        grid=(_B // _BM,),
        in_specs=[
            pl.BlockSpec((_BM, _TOTAL_BITS), lambda i: (i, 0)),
            pl.BlockSpec((_NUM_NEURONS, 16), lambda i: (0, 0)),
        ],
        out_specs=pl.BlockSpec((_NUM_NEURONS, _BM), lambda i: (0, i)),
        out_shape=jax.ShapeDtypeStruct((_NUM_NEURONS, _B), jnp.int32),
        scratch_shapes=[
            pltpu.VMEM((_NUM_NEURONS, _TOTAL_BITS), jnp.bfloat16),
            pltpu.VMEM((_NUM_NEURONS, _TOTAL_BITS), jnp.bfloat16),
            pltpu.VMEM((_NUM_NEURONS, 128), jnp.int32),
        ],
    )(input_bits, conn_p)


def _tr_body(in_ref, out_ref):
    out_ref[:] = in_ref[:].T


def _transpose_nb_to_bn(out_t):
    return pl.pallas_call(
        _tr_body,
        grid=(_B // _BM,),
        in_specs=[pl.BlockSpec((_NUM_NEURONS, _BM), lambda i: (0, i))],
        out_specs=pl.BlockSpec((_BM, _NUM_NEURONS), lambda i: (i, 0)),
        out_shape=jax.ShapeDtypeStruct((_B, _NUM_NEURONS), jnp.float32),
    )(out_t)


@functools.lru_cache(maxsize=1)
def _make_gather():
    mesh = plsc.VectorSubcoreMesh(core_axis_name="c", subcore_axis_name="s")

    @functools.partial(
        pl.kernel,
        mesh=mesh,
        out_type=jax.ShapeDtypeStruct((_FLAT,), jnp.float32),
        scratch_types=[
            pltpu.VMEM_SHARED((_SHW,), jnp.float32),
            pltpu.VMEM_SHARED((_SHW,), jnp.float32),
            pltpu.VMEM((_CH,), jnp.int32),
            pltpu.VMEM((_CH,), jnp.int32),
            pltpu.VMEM((_CH,), jnp.float32),
            pltpu.VMEM((_CH,), jnp.float32),
            pltpu.SemaphoreType.DMA,
            pltpu.SemaphoreType.DMA,
            pltpu.SemaphoreType.DMA,
            pltpu.SemaphoreType.DMA,
        ],
    )
    def gather_k(mem_hbm, idx_hbm, out_hbm, sh0, sh1, idx_v0, idx_v1,
                 val_v0, val_v1, gsem0, gsem1, wsem, ssem):
        s = lax.axis_index("s")
        c = lax.axis_index("c")
        core_slot_base = c * (64 * _SLOTS_SR)
        core_mem_base = c * (64 * _SR)
        sh_bufs = (sh0, sh1)
        idx_bufs = (idx_v0, idx_v1)
        val_bufs = (val_v0, val_v1)
        half = _CH // 2
        wbs = [None, None]
        stages = [None, None]

        def chunk_off(k):
            return core_slot_base + k * (_GRP * _SLOTS_SR) + s * _CH

        def start_stage(k):
            return pltpu.async_copy(
                mem_hbm.at[
                    pl.ds(core_mem_base + k * _SHW + s * (_SHW // 16),
                          _SHW // 16)
                ],
                sh_bufs[k % 2].at[pl.ds(s * (_SHW // 16), _SHW // 16)],
                ssem,
            )

        stages[0] = start_stage(0)
        pltpu.sync_copy(idx_hbm.at[pl.ds(chunk_off(0), _CH)], idx_v0)

        for k in range(_NPASS):
            cur = k % 2
            nxt = (k + 1) % 2
            stages[cur].wait()
            plsc.subcore_barrier()
            if k + 1 < _NPASS:
                stages[nxt] = start_stage(k + 1)
            sh = sh_bufs[cur]
            g0 = pltpu.async_copy(
                sh.at[idx_bufs[cur].at[pl.ds(0, half)]],
                val_bufs[cur].at[pl.ds(0, half)], gsem0,
            )
            g1 = pltpu.async_copy(
                sh.at[idx_bufs[cur].at[pl.ds(half, half)]],
                val_bufs[cur].at[pl.ds(half, half)], gsem1,
            )
            if k + 1 < _NPASS:
                pltpu.sync_copy(
                    idx_hbm.at[pl.ds(chunk_off(k + 1), _CH)], idx_bufs[nxt]
                )
                if wbs[nxt] is not None:
                    wbs[nxt].wait()
            g0.wait()
            g1.wait()
            wbs[cur] = pltpu.async_copy(
                val_bufs[cur], out_hbm.at[pl.ds(chunk_off(k), _CH)], wsem
            )
        wbs[0].wait()
        wbs[1].wait()

    return gather_k


def kernel(input_bits, connections, memory):
    conn_p = jnp.full((_NUM_NEURONS, 16), _TOTAL_BITS, jnp.int32)
    conn_p = conn_p.at[:, :_N_BITS].set(connections)
    idx_t = _addresses_t(input_bits, conn_p)
    mem_lin = (
        memory.reshape(_NUM_NEURONS // 8, 8, _MEM_SIZE // 128, 128)
        .transpose(0, 2, 1, 3)
        .reshape(_NUM_NEURONS * _MEM_SIZE)
    )
    idx_flat = (
        idx_t.reshape(_NUM_NEURONS // 8, 8, _B // 128, 128)
        .transpose(0, 2, 1, 3)
        .reshape(_FLAT)
    )
    out_flat = _make_gather()(mem_lin, idx_flat)
    out_t = (
        out_flat.reshape(_NUM_NEURONS // 8, _B // 128, 8, 128)
        .transpose(0, 2, 1, 3)
        .reshape(_NUM_NEURONS, _B)
    )
    return _transpose_nb_to_bn(out_t)
